# baseline (speedup 1.0000x reference)
import functools

import jax
import jax.numpy as jnp
from jax import lax
from jax.experimental import pallas as pl
from jax.experimental.pallas import tpu as pltpu
from jax.experimental.pallas import tpu_sc as plsc

N = 10000
NP = 10240
D = 256
HF = 128
E = 160000
TILES = 16
BLK = 64
BPT = 160
NB = 3
EP = TILES * BPT * BLK
CH = 8
NCH = BPT // CH
RPT = NP // TILES
ROWB = 512
GRID = NP // ROWB


_SC_MESH = plsc.VectorSubcoreMesh(core_axis_name="c", subcore_axis_name="s")

_SC_SCRATCH = [
    pltpu.VMEM((CH, BLK), jnp.int32),
    pltpu.VMEM((CH, BLK), jnp.int32),
    pltpu.VMEM((NB, BLK, HF), jnp.float32),
    pltpu.VMEM_SHARED((NP, HF), jnp.float32),
    pltpu.SemaphoreType.DMA,
    pltpu.SemaphoreType.DMA,
    pltpu.SemaphoreType.DMA,
]


@functools.partial(
    pl.kernel,
    out_type=jax.ShapeDtypeStruct((2 * NP, HF), jnp.float32),
    mesh=_SC_MESH,
    scratch_types=_SC_SCRATCH,
)
def _sc_agg(src_h, dst_h, tab_h, zrow_h,
            p_out, src_v, dst_v, rows_v, acc_sh, sem0, sem1, sem2):
    c = lax.axis_index("c")
    s = lax.axis_index("s")
    sems = (sem0, sem1, sem2)
    pltpu.sync_copy(zrow_h, acc_sh.at[pl.ds(s * RPT, RPT)])
    plsc.subcore_barrier()

    def chunk(k, carry):
        pltpu.sync_copy(src_h.at[(c * TILES + s) * NCH + k], src_v)
        pltpu.sync_copy(dst_h.at[s * NCH + k], dst_v)
        g = [None] * CH
        for b in range(2):
            g[b] = pltpu.async_copy(tab_h.at[src_v.at[b]],
                                    rows_v.at[b % NB], sems[b % NB])
        for b in range(CH):
            if b + 2 < CH:
                jj = (b + 2) % NB
                g[b + 2] = pltpu.async_copy(tab_h.at[src_v.at[b + 2]],
                                            rows_v.at[jj], sems[jj])
            g[b].wait()
            pltpu.sync_copy(rows_v.at[b % NB], acc_sh.at[dst_v.at[b]],
                            add=True)
        return carry

    lax.fori_loop(0, NCH, chunk, 0)
    plsc.subcore_barrier()
    pltpu.sync_copy(acc_sh.at[pl.ds(s * RPT, RPT)],
                    p_out.at[pl.ds(c * NP + s * RPT, RPT)])


@functools.partial(
    pl.kernel,
    out_type=jax.ShapeDtypeStruct((2 * NP, HF), jnp.float32),
    mesh=_SC_MESH,
    scratch_types=[
        pltpu.VMEM((CH, BLK), jnp.int32),
        pltpu.VMEM((BLK, HF), jnp.float32),
        pltpu.VMEM_SHARED((NP, HF), jnp.float32),
    ],
)
def _sc_deg(dst_h, ones_h, zrow_h, deg_out, dst_v, ones_v, acc_sh):
    c = lax.axis_index("c")
    s = lax.axis_index("s")
    pltpu.sync_copy(ones_h, ones_v)
    pltpu.sync_copy(zrow_h, acc_sh.at[pl.ds(s * RPT, RPT)])
    plsc.subcore_barrier()

    def chunk(k, carry):
        pltpu.sync_copy(dst_h.at[s * NCH + k], dst_v)

        def burst(b, carry2):
            pltpu.sync_copy(ones_v, acc_sh.at[dst_v.at[b]], add=True)
            return carry2

        return lax.fori_loop(0, CH, burst, carry)

    lax.fori_loop(0, NCH, chunk, 0)
    plsc.subcore_barrier()
    pltpu.sync_copy(acc_sh.at[pl.ds(s * RPT, RPT)],
                    deg_out.at[pl.ds(c * NP + s * RPT, RPT)])


def _mm_first_body(h_ref, w_ref, b_ref, s_ref, z_ref):
    o = jnp.dot(h_ref[...], w_ref[...], preferred_element_type=jnp.float32)
    s_ref[...] = o[:, :D] + b_ref[...]
    z_ref[0] = o[:, D:D + HF]
    z_ref[1] = o[:, D + HF:]


def _mm_mid_body(s_ref, p_ref, deg_ref, w_ref, b_ref, so_ref, z_ref):
    deg = jnp.maximum(deg_ref[:, 0:1], 1.0)
    pcat = jnp.concatenate([p_ref[0], p_ref[1]], axis=-1)
    h = jnp.maximum(s_ref[...] + pcat / deg, 0.0)
    o = jnp.dot(h, w_ref[...], preferred_element_type=jnp.float32)
    so_ref[...] = o[:, :D] + b_ref[...]
    z_ref[0] = o[:, D:D + HF]
    z_ref[1] = o[:, D + HF:]


def _final_body(s_ref, p_ref, deg_ref, o_ref):
    deg = jnp.maximum(deg_ref[:, 0:1], 1.0)
    pcat = jnp.concatenate([p_ref[0], p_ref[1]], axis=-1)
    o_ref[...] = s_ref[...] + pcat / deg


_spec_rows = pl.BlockSpec((ROWB, D), lambda i: (i, 0))
_spec_p = pl.BlockSpec((2, ROWB, HF), lambda i: (0, i, 0))
_spec_deg = pl.BlockSpec((ROWB, HF), lambda i: (i, 0))
_spec_w = pl.BlockSpec((D, 2 * D), lambda i: (0, 0))
_spec_b = pl.BlockSpec((1, D), lambda i: (0, 0))
_out_sz = [jax.ShapeDtypeStruct((NP, D), jnp.float32),
           jax.ShapeDtypeStruct((2, NP, HF), jnp.float32)]

_mm_first = pl.pallas_call(
    _mm_first_body, grid=(GRID,),
    in_specs=[_spec_rows, _spec_w, _spec_b],
    out_specs=[_spec_rows, _spec_p],
    out_shape=_out_sz,
)

_mm_mid = pl.pallas_call(
    _mm_mid_body, grid=(GRID,),
    in_specs=[_spec_rows, _spec_p, _spec_deg, _spec_w, _spec_b],
    out_specs=[_spec_rows, _spec_p],
    out_shape=_out_sz,
)

_final = pl.pallas_call(
    _final_body, grid=(GRID,),
    in_specs=[_spec_rows, _spec_p, _spec_deg],
    out_specs=_spec_rows,
    out_shape=jax.ShapeDtypeStruct((NP, D), jnp.float32),
)


def kernel(features, edge_index, Ws0, Wn0, b0, Ws1, Wn1, b1, Ws2, Wn2, b2):
    src = edge_index[0].astype(jnp.int32)
    dst = edge_index[1].astype(jnp.int32)
    npad = EP - E
    src_p = jnp.concatenate([src, jnp.zeros((npad,), jnp.int32)])
    dst_p = jnp.concatenate(
        [dst, N + (jnp.arange(npad, dtype=jnp.int32) % (NP - N))])
    src4 = jnp.stack([src_p, src_p + NP]).reshape(2 * TILES * NCH, CH, BLK)
    dst3 = dst_p.reshape(TILES * NCH, CH, BLK)
    zrow = jnp.zeros((RPT, HF), jnp.float32)
    ones_blk = jnp.ones((BLK, HF), jnp.float32)
    hpad = jnp.concatenate(
        [features, jnp.zeros((NP - N, D), jnp.float32)])

    Wc0 = jnp.concatenate([Ws0, Wn0], axis=1)
    Wc1 = jnp.concatenate([Ws1, Wn1], axis=1)
    Wc2 = jnp.concatenate([Ws2, Wn2], axis=1)

    deg = _sc_deg(dst3, ones_blk, zrow)[:NP]

    S0, Z0 = _mm_first(hpad, Wc0, b0.reshape(1, D))
    P0 = _sc_agg(src4, dst3, Z0.reshape(2 * NP, HF), zrow)
    S1, Z1 = _mm_mid(S0, P0.reshape(2, NP, HF), deg, Wc1, b1.reshape(1, D))
    P1 = _sc_agg(src4, dst3, Z1.reshape(2 * NP, HF), zrow)
    S2, Z2 = _mm_mid(S1, P1.reshape(2, NP, HF), deg, Wc2, b2.reshape(1, D))
    P2 = _sc_agg(src4, dst3, Z2.reshape(2 * NP, HF), zrow)
    out = _final(S2, P2.reshape(2, NP, HF), deg)
    return out[:N]


# speedup vs baseline: 1.3044x; 1.3044x over previous
import functools

import jax
import jax.numpy as jnp
from jax import lax
from jax.experimental import pallas as pl
from jax.experimental.pallas import tpu as pltpu
from jax.experimental.pallas import tpu_sc as plsc

N = 10000
NP = 10240
D = 256
HF = 128
E = 160000
TILES = 16
BLK = 128
BPT = 80
NB = 2
EP = TILES * BPT * BLK
CH = 40
NCH = BPT // CH
RPT = NP // TILES
ROWB = 512
GRID = NP // ROWB


_SC_MESH = plsc.VectorSubcoreMesh(core_axis_name="c", subcore_axis_name="s")

_SC_SCRATCH = [
    pltpu.VMEM((CH, BLK), jnp.int32),
    pltpu.VMEM((CH, BLK), jnp.int32),
    pltpu.VMEM((NB, BLK, HF), jnp.float32),
    pltpu.VMEM_SHARED((NP, HF), jnp.float32),
    pltpu.SemaphoreType.DMA,
    pltpu.SemaphoreType.DMA,
]


@functools.partial(
    pl.kernel,
    out_type=jax.ShapeDtypeStruct((2 * NP, HF), jnp.float32),
    mesh=_SC_MESH,
    scratch_types=_SC_SCRATCH,
)
def _sc_agg(src_h, dst_h, tab_h, zrow_h,
            p_out, src_v, dst_v, rows_v, acc_sh, sem0, sem1):
    c = lax.axis_index("c")
    s = lax.axis_index("s")
    sems = (sem0, sem1)
    pltpu.sync_copy(zrow_h, acc_sh.at[pl.ds(s * RPT, RPT)])
    plsc.subcore_barrier()

    def chunk(k, carry):
        pltpu.sync_copy(src_h.at[(c * TILES + s) * NCH + k], src_v)
        pltpu.sync_copy(dst_h.at[s * NCH + k], dst_v)
        g = [None] * CH
        g[0] = pltpu.async_copy(tab_h.at[src_v.at[0]],
                                rows_v.at[0], sems[0])
        for b in range(CH):
            if b + 1 < CH:
                jj = (b + 1) % NB
                g[b + 1] = pltpu.async_copy(tab_h.at[src_v.at[b + 1]],
                                            rows_v.at[jj], sems[jj])
            g[b].wait()
            pltpu.sync_copy(rows_v.at[b % NB], acc_sh.at[dst_v.at[b]],
                            add=True)
        return carry

    lax.fori_loop(0, NCH, chunk, 0)
    plsc.subcore_barrier()
    pltpu.sync_copy(acc_sh.at[pl.ds(s * RPT, RPT)],
                    p_out.at[pl.ds(c * NP + s * RPT, RPT)])


@functools.partial(
    pl.kernel,
    out_type=jax.ShapeDtypeStruct((2 * NP, HF), jnp.float32),
    mesh=_SC_MESH,
    scratch_types=[
        pltpu.VMEM((CH, BLK), jnp.int32),
        pltpu.VMEM((BLK, HF), jnp.float32),
        pltpu.VMEM_SHARED((NP, HF), jnp.float32),
    ],
)
def _sc_deg(dst_h, ones_h, zrow_h, deg_out, dst_v, ones_v, acc_sh):
    c = lax.axis_index("c")
    s = lax.axis_index("s")
    pltpu.sync_copy(ones_h, ones_v)
    pltpu.sync_copy(zrow_h, acc_sh.at[pl.ds(s * RPT, RPT)])
    plsc.subcore_barrier()

    def chunk(k, carry):
        pltpu.sync_copy(dst_h.at[s * NCH + k], dst_v)

        def burst(b, carry2):
            pltpu.sync_copy(ones_v, acc_sh.at[dst_v.at[b]], add=True)
            return carry2

        return lax.fori_loop(0, CH, burst, carry)

    lax.fori_loop(0, NCH, chunk, 0)
    plsc.subcore_barrier()
    pltpu.sync_copy(acc_sh.at[pl.ds(s * RPT, RPT)],
                    deg_out.at[pl.ds(c * NP + s * RPT, RPT)])


def _mm_first_body(h_ref, w_ref, b_ref, s_ref, z_ref):
    o = jnp.dot(h_ref[...], w_ref[...], preferred_element_type=jnp.float32)
    s_ref[...] = o[:, :D] + b_ref[...]
    z_ref[0] = o[:, D:D + HF]
    z_ref[1] = o[:, D + HF:]


def _mm_mid_body(s_ref, p_ref, deg_ref, w_ref, b_ref, so_ref, z_ref):
    deg = jnp.maximum(deg_ref[:, 0:1], 1.0)
    pcat = jnp.concatenate([p_ref[0], p_ref[1]], axis=-1)
    h = jnp.maximum(s_ref[...] + pcat / deg, 0.0)
    o = jnp.dot(h, w_ref[...], preferred_element_type=jnp.float32)
    so_ref[...] = o[:, :D] + b_ref[...]
    z_ref[0] = o[:, D:D + HF]
    z_ref[1] = o[:, D + HF:]


def _final_body(s_ref, p_ref, deg_ref, o_ref):
    deg = jnp.maximum(deg_ref[:, 0:1], 1.0)
    pcat = jnp.concatenate([p_ref[0], p_ref[1]], axis=-1)
    o_ref[...] = s_ref[...] + pcat / deg


_spec_rows = pl.BlockSpec((ROWB, D), lambda i: (i, 0))
_spec_p = pl.BlockSpec((2, ROWB, HF), lambda i: (0, i, 0))
_spec_deg = pl.BlockSpec((ROWB, HF), lambda i: (i, 0))
_spec_w = pl.BlockSpec((D, 2 * D), lambda i: (0, 0))
_spec_b = pl.BlockSpec((1, D), lambda i: (0, 0))
_out_sz = [jax.ShapeDtypeStruct((NP, D), jnp.float32),
           jax.ShapeDtypeStruct((2, NP, HF), jnp.float32)]

_mm_first = pl.pallas_call(
    _mm_first_body, grid=(GRID,),
    in_specs=[_spec_rows, _spec_w, _spec_b],
    out_specs=[_spec_rows, _spec_p],
    out_shape=_out_sz,
)

_mm_mid = pl.pallas_call(
    _mm_mid_body, grid=(GRID,),
    in_specs=[_spec_rows, _spec_p, _spec_deg, _spec_w, _spec_b],
    out_specs=[_spec_rows, _spec_p],
    out_shape=_out_sz,
)

_final = pl.pallas_call(
    _final_body, grid=(GRID,),
    in_specs=[_spec_rows, _spec_p, _spec_deg],
    out_specs=_spec_rows,
    out_shape=jax.ShapeDtypeStruct((NP, D), jnp.float32),
)


def kernel(features, edge_index, Ws0, Wn0, b0, Ws1, Wn1, b1, Ws2, Wn2, b2):
    src = edge_index[0].astype(jnp.int32)
    dst = edge_index[1].astype(jnp.int32)
    npad = EP - E
    src_p = jnp.concatenate([src, jnp.zeros((npad,), jnp.int32)])
    dst_p = jnp.concatenate(
        [dst, N + (jnp.arange(npad, dtype=jnp.int32) % (NP - N))])
    src4 = jnp.stack([src_p, src_p + NP]).reshape(2 * TILES * NCH, CH, BLK)
    dst3 = dst_p.reshape(TILES * NCH, CH, BLK)
    zrow = jnp.zeros((RPT, HF), jnp.float32)
    ones_blk = jnp.ones((BLK, HF), jnp.float32)
    hpad = jnp.concatenate(
        [features, jnp.zeros((NP - N, D), jnp.float32)])

    Wc0 = jnp.concatenate([Ws0, Wn0], axis=1)
    Wc1 = jnp.concatenate([Ws1, Wn1], axis=1)
    Wc2 = jnp.concatenate([Ws2, Wn2], axis=1)

    deg = _sc_deg(dst3, ones_blk, zrow)[:NP]

    S0, Z0 = _mm_first(hpad, Wc0, b0.reshape(1, D))
    P0 = _sc_agg(src4, dst3, Z0.reshape(2 * NP, HF), zrow)
    S1, Z1 = _mm_mid(S0, P0.reshape(2, NP, HF), deg, Wc1, b1.reshape(1, D))
    P1 = _sc_agg(src4, dst3, Z1.reshape(2 * NP, HF), zrow)
    S2, Z2 = _mm_mid(S1, P1.reshape(2, NP, HF), deg, Wc2, b2.reshape(1, D))
    P2 = _sc_agg(src4, dst3, Z2.reshape(2 * NP, HF), zrow)
    out = _final(S2, P2.reshape(2, NP, HF), deg)
    return out[:N]
